# baseline (speedup 1.0000x reference)
import math
import functools

import jax
import jax.numpy as jnp
from jax.experimental import pallas as pl
from jax.experimental.pallas import tpu as pltpu

LAMDA = 0.5
ALPHA = 0.1
_AB = ALPHA / (1.0 - ALPHA)


def _eye(n, dtype):
    r = jax.lax.broadcasted_iota(jnp.int32, (n, n), 0)
    c = jax.lax.broadcasted_iota(jnp.int32, (n, n), 1)
    return jnp.where(r == c, jnp.ones((), dtype), jnp.zeros((), dtype))


def _stage0_body(x_ref, w1_ref, b1_ref, wc0_ref, h0bf_ref, a1_ref, *, theta0):
    nhidden = w1_ref.shape[1]
    h0 = jnp.maximum(
        jnp.dot(x_ref[...], w1_ref[...],
                preferred_element_type=jnp.float32) + b1_ref[...], 0.0)
    h0bf_ref[...] = h0.astype(jnp.bfloat16)
    m = ((1.0 - ALPHA) * theta0) * wc0_ref[...] \
        + ((1.0 - ALPHA) * (1.0 - theta0)) * _eye(nhidden, jnp.float32)
    a1_ref[...] = jnp.dot(h0, m, preferred_element_type=jnp.float32)


def _stage1_body(g_ref, a1_ref, gp_ref, h1_ref, *, bm):
    i = pl.program_id(0)
    g = g_ref[...]
    gp_ref[0] = pltpu.bitcast(g.astype(jnp.bfloat16), jnp.float32)
    hi = jnp.dot(g, a1_ref[...], preferred_element_type=jnp.float32)
    row0 = pl.multiple_of(i * bm, 16)
    h1_ref[...] = jnp.maximum(
        hi + _AB * a1_ref[pl.ds(row0, bm), :], 0.0).astype(jnp.bfloat16)


def _stage2_body(gp_ref, h1_ref, h0bf_ref, wc_ref, w2_ref, b2_ref,
                 out_ref, hb_ref, a_ref, b_ref, *, bm, nrest, thetas):
    l = pl.program_id(0)
    i = pl.program_id(1)
    nhidden = a_ref.shape[1]

    @pl.when(i == 0)
    def _layer_setup():
        @pl.when(l == 0)
        def _seed():
            hb_ref[1] = h1_ref[...]

        theta = thetas[0]
        for k in range(1, nrest):
            theta = jnp.where(l == k, thetas[k], theta)
        m = ((1.0 - ALPHA) * theta) * wc_ref[0] \
            + ((1.0 - ALPHA) * (1.0 - theta)) * _eye(nhidden, jnp.float32)
        mbf = m.astype(jnp.bfloat16)
        h_cur = hb_ref[(l + 1) % 2]
        a = jnp.dot(h_cur, mbf, preferred_element_type=jnp.float32)
        a_ref[...] = a.astype(jnp.bfloat16)
        b_ref[...] = _AB * jnp.dot(
            h0bf_ref[...], mbf, preferred_element_type=jnp.float32)

    g = pltpu.bitcast(gp_ref[0], jnp.bfloat16)
    hi = jnp.dot(g, a_ref[...], preferred_element_type=jnp.float32)
    row0 = pl.multiple_of(i * bm, 16)
    hnew = jnp.maximum(hi + b_ref[pl.ds(row0, bm), :], 0.0)
    hb_ref[l % 2, pl.ds(row0, bm), :] = hnew.astype(jnp.bfloat16)

    @pl.when(l == nrest - 1)
    def _proj():
        out_ref[...] = (jnp.dot(hnew.astype(jnp.bfloat16),
                                w2_ref[...].astype(jnp.bfloat16),
                                preferred_element_type=jnp.float32)
                        + b2_ref[...])


@jax.jit
def kernel(input, adj, G, W1, b1, Wc, W2, b2):
    del adj
    n, nfeat = input.shape
    nhidden = W1.shape[1]
    nclass = W2.shape[1]
    nlayers = Wc.shape[0]
    bm1 = 400 if n % 400 == 0 else n
    chunk = 200 if n % 400 == 0 else n // 2
    nchunks = (n // 2) // chunk
    thetas = tuple(math.log(LAMDA / (k + 1) + 1.0) for k in range(nlayers))

    s0 = functools.partial(_stage0_body, theta0=thetas[0])
    h0bf, a1 = pl.pallas_call(
        s0,
        out_shape=[
            jax.ShapeDtypeStruct((n, nhidden), jnp.bfloat16),
            jax.ShapeDtypeStruct((n, nhidden), jnp.float32),
        ],
    )(input, W1, b1.reshape(1, -1), Wc[0])

    s1 = functools.partial(_stage1_body, bm=bm1)
    pk1 = bm1 // 2
    gp, h1 = pl.pallas_call(
        s1,
        grid=(n // bm1,),
        in_specs=[
            pl.BlockSpec((bm1, n), lambda i: (i, 0)),
            pl.BlockSpec((n, nhidden), lambda i: (0, 0)),
        ],
        out_specs=[
            pl.BlockSpec((1, pk1, n),
                         lambda i, _c=chunk, _p=pk1: (i * _p // _c,
                                                      (i * _p % _c) // _p, 0)),
            pl.BlockSpec((bm1, nhidden), lambda i: (i, 0)),
        ],
        out_shape=[
            jax.ShapeDtypeStruct((nchunks, chunk, n), jnp.float32),
            jax.ShapeDtypeStruct((n, nhidden), jnp.bfloat16),
        ],
    )(G, a1)

    nrest = nlayers - 1
    bm2 = 2 * chunk
    s2 = functools.partial(_stage2_body, bm=bm2, nrest=nrest,
                           thetas=thetas[1:])
    out = pl.pallas_call(
        s2,
        grid=(nrest, nchunks),
        in_specs=[
            pl.BlockSpec((1, chunk, n), lambda l, i: (i, 0, 0)),
            pl.BlockSpec((n, nhidden), lambda l, i: (0, 0)),
            pl.BlockSpec((n, nhidden), lambda l, i: (0, 0)),
            pl.BlockSpec((1, nhidden, nhidden), lambda l, i: (l + 1, 0, 0)),
            pl.BlockSpec((nhidden, nclass), lambda l, i: (0, 0)),
            pl.BlockSpec((1, nclass), lambda l, i: (0, 0)),
        ],
        out_specs=pl.BlockSpec((bm2, nclass), lambda l, i: (i, 0)),
        out_shape=jax.ShapeDtypeStruct((n, nclass), jnp.float32),
        scratch_shapes=[
            pltpu.VMEM((2, n, nhidden), jnp.bfloat16),
            pltpu.VMEM((n, nhidden), jnp.bfloat16),
            pltpu.VMEM((n, nhidden), jnp.float32),
        ],
    )(gp, h1, h0bf, Wc, W2, b2.reshape(1, -1))
    return out


# speedup vs baseline: 1.0625x; 1.0625x over previous
import math
import functools

import jax
import jax.numpy as jnp
from jax.experimental import pallas as pl
from jax.experimental.pallas import tpu as pltpu

LAMDA = 0.5
ALPHA = 0.1
_AB = ALPHA / (1.0 - ALPHA)


def _eye(n, dtype):
    r = jax.lax.broadcasted_iota(jnp.int32, (n, n), 0)
    c = jax.lax.broadcasted_iota(jnp.int32, (n, n), 1)
    return jnp.where(r == c, jnp.ones((), dtype), jnp.zeros((), dtype))


def _stage0_body(x_ref, w1_ref, b1_ref, wc0_ref, h0bf_ref, a1_ref, *, theta0):
    nhidden = w1_ref.shape[1]
    h0 = jnp.maximum(
        jnp.dot(x_ref[...], w1_ref[...],
                preferred_element_type=jnp.float32) + b1_ref[...], 0.0)
    h0bf_ref[...] = h0.astype(jnp.bfloat16)
    m = ((1.0 - ALPHA) * theta0) * wc0_ref[...] \
        + ((1.0 - ALPHA) * (1.0 - theta0)) * _eye(nhidden, jnp.float32)
    a1_ref[...] = jnp.dot(h0, m, preferred_element_type=jnp.float32)


def _stage1_body(g_ref, a1_ref, gbf_ref, h1_ref, *, bm):
    i = pl.program_id(0)
    g = g_ref[...]
    gbf_ref[...] = g.astype(jnp.bfloat16)
    hi = jnp.dot(g, a1_ref[...], preferred_element_type=jnp.float32)
    row0 = pl.multiple_of(i * bm, 16)
    h1_ref[...] = jnp.maximum(
        hi + _AB * a1_ref[pl.ds(row0, bm), :], 0.0).astype(jnp.bfloat16)


def _stage2_body(gbf_ref, h1_ref, h0bf_ref, wc_ref, w2_ref, b2_ref,
                 out_ref, hb_ref, a_ref, b_ref, *, bm, nrest, thetas, ksplit):
    l = pl.program_id(0)
    i = pl.program_id(1)
    nhidden = a_ref.shape[1]

    @pl.when(i == 0)
    def _layer_setup():
        @pl.when(l == 0)
        def _seed():
            hb_ref[1] = h1_ref[...]

        theta = thetas[0]
        for k in range(1, nrest):
            theta = jnp.where(l == k, thetas[k], theta)
        m = ((1.0 - ALPHA) * theta) * wc_ref[0] \
            + ((1.0 - ALPHA) * (1.0 - theta)) * _eye(nhidden, jnp.float32)
        mbf = m.astype(jnp.bfloat16)
        h_cur = hb_ref[(l + 1) % 2]
        a = jnp.dot(h_cur, mbf, preferred_element_type=jnp.float32)
        a_ref[...] = a.astype(jnp.bfloat16)
        b_ref[...] = _AB * jnp.dot(
            h0bf_ref[...], mbf, preferred_element_type=jnp.float32)

    g = gbf_ref[...]
    a = a_ref[...]
    hi = (jnp.dot(g[:, :ksplit], a[:ksplit, :],
                  preferred_element_type=jnp.float32)
          + jnp.dot(g[:, ksplit:], a[ksplit:, :],
                    preferred_element_type=jnp.float32))
    row0 = pl.multiple_of(i * bm, 16)
    hnew = jnp.maximum(hi + b_ref[pl.ds(row0, bm), :], 0.0)
    hb_ref[l % 2, pl.ds(row0, bm), :] = hnew.astype(jnp.bfloat16)

    @pl.when(l == nrest - 1)
    def _proj():
        out_ref[...] = (jnp.dot(hnew.astype(jnp.bfloat16),
                                w2_ref[...].astype(jnp.bfloat16),
                                preferred_element_type=jnp.float32)
                        + b2_ref[...])


@jax.jit
def kernel(input, adj, G, W1, b1, Wc, W2, b2):
    del adj
    n, nfeat = input.shape
    nhidden = W1.shape[1]
    nclass = W2.shape[1]
    nlayers = Wc.shape[0]
    bm1 = 400 if n % 400 == 0 else n
    bm2 = 1000 if n % 1000 == 0 else n
    ksplit = (n // 2 // 512) * 512 if n >= 1024 else n // 2
    thetas = tuple(math.log(LAMDA / (k + 1) + 1.0) for k in range(nlayers))

    s0 = functools.partial(_stage0_body, theta0=thetas[0])
    h0bf, a1 = pl.pallas_call(
        s0,
        out_shape=[
            jax.ShapeDtypeStruct((n, nhidden), jnp.bfloat16),
            jax.ShapeDtypeStruct((n, nhidden), jnp.float32),
        ],
    )(input, W1, b1.reshape(1, -1), Wc[0])

    s1 = functools.partial(_stage1_body, bm=bm1)
    gbf, h1 = pl.pallas_call(
        s1,
        grid=(n // bm1,),
        in_specs=[
            pl.BlockSpec((bm1, n), lambda i: (i, 0)),
            pl.BlockSpec((n, nhidden), lambda i: (0, 0)),
        ],
        out_specs=[
            pl.BlockSpec((bm1, n), lambda i: (i, 0)),
            pl.BlockSpec((bm1, nhidden), lambda i: (i, 0)),
        ],
        out_shape=[
            jax.ShapeDtypeStruct((n, n), jnp.bfloat16),
            jax.ShapeDtypeStruct((n, nhidden), jnp.bfloat16),
        ],
    )(G, a1)

    nrest = nlayers - 1
    s2 = functools.partial(_stage2_body, bm=bm2, nrest=nrest,
                           thetas=thetas[1:], ksplit=ksplit)
    out = pl.pallas_call(
        s2,
        grid=(nrest, n // bm2),
        in_specs=[
            pl.BlockSpec((bm2, n), lambda l, i: (i, 0)),
            pl.BlockSpec((n, nhidden), lambda l, i: (0, 0)),
            pl.BlockSpec((n, nhidden), lambda l, i: (0, 0)),
            pl.BlockSpec((1, nhidden, nhidden), lambda l, i: (l + 1, 0, 0)),
            pl.BlockSpec((nhidden, nclass), lambda l, i: (0, 0)),
            pl.BlockSpec((1, nclass), lambda l, i: (0, 0)),
        ],
        out_specs=pl.BlockSpec((bm2, nclass), lambda l, i: (i, 0)),
        out_shape=jax.ShapeDtypeStruct((n, nclass), jnp.float32),
        scratch_shapes=[
            pltpu.VMEM((2, n, nhidden), jnp.bfloat16),
            pltpu.VMEM((n, nhidden), jnp.bfloat16),
            pltpu.VMEM((n, nhidden), jnp.float32),
        ],
    )(gbf, h1, h0bf, Wc, W2, b2.reshape(1, -1))
    return out


# speedup vs baseline: 1.0860x; 1.0222x over previous
import math
import functools

import jax
import jax.numpy as jnp
from jax.experimental import pallas as pl
from jax.experimental.pallas import tpu as pltpu

LAMDA = 0.5
ALPHA = 0.1


def _stage1_body(x_ref, g_ref, w1_ref, b1_ref, wc0_ref,
                 gbf_ref, h1_ref, h0_ref, *, bm, theta0):
    i = pl.program_id(0)

    @pl.when(i == 0)
    def _init():
        h0_ref[...] = jnp.maximum(
            jnp.dot(x_ref[...], w1_ref[...],
                    preferred_element_type=jnp.float32) + b1_ref[...], 0.0)

    g = g_ref[...]
    gbf_ref[...] = g.astype(jnp.bfloat16)
    hi = jnp.dot(g, h0_ref[...], preferred_element_type=jnp.float32)
    row0 = pl.multiple_of(i * bm, 16)
    support = (1.0 - ALPHA) * hi + ALPHA * h0_ref[pl.ds(row0, bm), :]
    h1_ref[...] = jnp.maximum(
        theta0 * jnp.dot(support, wc0_ref[0],
                         preferred_element_type=jnp.float32)
        + (1.0 - theta0) * support, 0.0)


def _stage2_body(gbf_ref, h1_ref, h0_ref, wc_ref, w2_ref, b2_ref,
                 out_ref, hb_ref, *, bm, nrest, thetas):
    l = pl.program_id(0)
    i = pl.program_id(1)

    @pl.when((l == 0) & (i == 0))
    def _init():
        hb_ref[1] = h1_ref[...].astype(jnp.bfloat16)

    row0 = pl.multiple_of(i * bm, 16)
    h_cur = hb_ref[(l + 1) % 2]
    hi = jnp.dot(gbf_ref[...], h_cur, preferred_element_type=jnp.float32)
    support = (1.0 - ALPHA) * hi + ALPHA * h0_ref[pl.ds(row0, bm), :]

    theta = thetas[0]
    for k in range(1, nrest):
        theta = jnp.where(l == k, thetas[k], theta)

    hnew = jnp.maximum(
        theta * jnp.dot(support, wc_ref[0],
                        preferred_element_type=jnp.float32)
        + (1.0 - theta) * support, 0.0)
    hb_ref[l % 2, pl.ds(row0, bm), :] = hnew.astype(jnp.bfloat16)

    @pl.when(l == nrest - 1)
    def _proj():
        out_ref[...] = (jnp.dot(hnew, w2_ref[...],
                                preferred_element_type=jnp.float32)
                        + b2_ref[...])


@jax.jit
def kernel(input, adj, G, W1, b1, Wc, W2, b2):
    del adj
    n, nfeat = input.shape
    nhidden = W1.shape[1]
    nclass = W2.shape[1]
    nlayers = Wc.shape[0]
    bm1 = 400 if n % 400 == 0 else n
    bm2 = 1000 if n % 1000 == 0 else n
    thetas = tuple(math.log(LAMDA / (k + 1) + 1.0) for k in range(nlayers))

    s1 = functools.partial(_stage1_body, bm=bm1, theta0=thetas[0])
    gbf, h1, h0 = pl.pallas_call(
        s1,
        grid=(n // bm1,),
        in_specs=[
            pl.BlockSpec((n, nfeat), lambda i: (0, 0)),
            pl.BlockSpec((bm1, n), lambda i: (i, 0)),
            pl.BlockSpec((nfeat, nhidden), lambda i: (0, 0)),
            pl.BlockSpec((1, nhidden), lambda i: (0, 0)),
            pl.BlockSpec((1, nhidden, nhidden), lambda i: (0, 0, 0)),
        ],
        out_specs=[
            pl.BlockSpec((bm1, n), lambda i: (i, 0)),
            pl.BlockSpec((bm1, nhidden), lambda i: (i, 0)),
            pl.BlockSpec((n, nhidden), lambda i: (0, 0)),
        ],
        out_shape=[
            jax.ShapeDtypeStruct((n, n), jnp.bfloat16),
            jax.ShapeDtypeStruct((n, nhidden), jnp.float32),
            jax.ShapeDtypeStruct((n, nhidden), jnp.float32),
        ],
    )(input, G, W1, b1.reshape(1, -1), Wc)

    nrest = nlayers - 1
    s2 = functools.partial(_stage2_body, bm=bm2, nrest=nrest,
                           thetas=thetas[1:])
    out = pl.pallas_call(
        s2,
        grid=(nrest, n // bm2),
        in_specs=[
            pl.BlockSpec((bm2, n), lambda l, i: (i, 0)),
            pl.BlockSpec((n, nhidden), lambda l, i: (0, 0)),
            pl.BlockSpec((n, nhidden), lambda l, i: (0, 0)),
            pl.BlockSpec((1, nhidden, nhidden), lambda l, i: (l + 1, 0, 0)),
            pl.BlockSpec((nhidden, nclass), lambda l, i: (0, 0)),
            pl.BlockSpec((1, nclass), lambda l, i: (0, 0)),
        ],
        out_specs=pl.BlockSpec((bm2, nclass), lambda l, i: (i, 0)),
        out_shape=jax.ShapeDtypeStruct((n, nclass), jnp.float32),
        scratch_shapes=[
            pltpu.VMEM((2, n, nhidden), jnp.bfloat16),
        ],
    )(gbf, h1, h0, Wc, W2, b2.reshape(1, -1))
    return out
